# baseline (speedup 1.0000x reference)
import functools

import jax
import jax.numpy as jnp
from jax.experimental import pallas as pl

N_ADDR = 1000000
D = 64
TEMPERATURE = 0.1
EPS = 1e-8

BLOCK = 8000


def _body(q_ref, a_ref, wsum_ref, ssum_ref):
    i = pl.program_id(0)

    @pl.when(i == 0)
    def _init():
        wsum_ref[...] = jnp.zeros_like(wsum_ref)
        ssum_ref[...] = jnp.zeros_like(ssum_ref)

    a = a_ref[...]
    q = q_ref[...]
    dots = jax.lax.dot_general(
        a, q, (((1,), (1,)), ((), ())),
        preferred_element_type=jnp.float32)
    sumsq = jnp.sum(a * a, axis=1, keepdims=True)
    qn = jnp.sqrt(jnp.sum(q * q))
    an = jnp.sqrt(sumsq)
    sim = dots / jnp.maximum(an * qn, EPS)
    w = jnp.exp((sim - 1.0) / TEMPERATURE)
    part = jax.lax.dot_general(
        w, a, (((0,), (0,)), ((), ())),
        preferred_element_type=jnp.float32)
    wsum_ref[...] += part
    ssum_ref[...] += jnp.sum(w)


@jax.jit
def kernel(query_address, addresses):
    grid = N_ADDR // BLOCK
    wsum, ssum = pl.pallas_call(
        _body,
        grid=(grid,),
        in_specs=[
            pl.BlockSpec((1, D), lambda i: (0, 0)),
            pl.BlockSpec((BLOCK, D), lambda i: (i, 0)),
        ],
        out_specs=[
            pl.BlockSpec((1, D), lambda i: (0, 0)),
            pl.BlockSpec((1, 1), lambda i: (0, 0)),
        ],
        out_shape=[
            jax.ShapeDtypeStruct((1, D), jnp.float32),
            jax.ShapeDtypeStruct((1, 1), jnp.float32),
        ],
    )(query_address, addresses)
    return (wsum[0] / ssum[0, 0]).reshape(-1)


# speedup vs baseline: 1.2008x; 1.2008x over previous
import jax
import jax.numpy as jnp
from jax.experimental import pallas as pl

N_ADDR = 1000000
D = 64
TEMPERATURE = 0.1
EPS = 1e-8

BLOCK2 = 5000


def _body(q_ref, lq_ref, a_ref, wsum_ref, ssum_ref):
    i = pl.program_id(0)

    @pl.when(i == 0)
    def _init():
        wsum_ref[...] = jnp.zeros_like(wsum_ref)
        ssum_ref[...] = jnp.zeros_like(ssum_ref)

    a2 = a_ref[...]
    lq = lq_ref[...]
    q = q_ref[...]

    dots2 = jax.lax.dot_general(
        lq, a2, (((1,), (1,)), ((), ())),
        preferred_element_type=jnp.float32)
    lane = jax.lax.broadcasted_iota(jnp.int32, (2, 128), 1)
    row = jax.lax.broadcasted_iota(jnp.int32, (2, 128), 0)
    lones = (lane // D == row).astype(jnp.float32)
    sumsq2 = jax.lax.dot_general(
        lones, a2 * a2, (((1,), (1,)), ((), ())),
        preferred_element_type=jnp.float32)

    qn = jnp.sqrt(jnp.sum(q * q))
    an = jnp.sqrt(sumsq2)
    sim = dots2 / jnp.maximum(an * qn, EPS)
    w = jnp.exp((sim - 1.0) / TEMPERATURE)

    part = jax.lax.dot_general(
        w, a2, (((1,), (0,)), ((), ())),
        preferred_element_type=jnp.float32)
    wsum_ref[...] += part
    ssum_ref[...] += jnp.sum(w)


@jax.jit
def kernel(query_address, addresses):
    n2 = N_ADDR // 2
    a2 = addresses.reshape(n2, 2 * D)
    z = jnp.zeros((1, D), jnp.float32)
    lq = jnp.concatenate(
        [jnp.concatenate([query_address, z], axis=1),
         jnp.concatenate([z, query_address], axis=1)], axis=0)
    grid = n2 // BLOCK2
    wsum, ssum = pl.pallas_call(
        _body,
        grid=(grid,),
        in_specs=[
            pl.BlockSpec((1, D), lambda i: (0, 0)),
            pl.BlockSpec((2, 2 * D), lambda i: (0, 0)),
            pl.BlockSpec((BLOCK2, 2 * D), lambda i: (i, 0)),
        ],
        out_specs=[
            pl.BlockSpec((2, 2 * D), lambda i: (0, 0)),
            pl.BlockSpec((1, 1), lambda i: (0, 0)),
        ],
        out_shape=[
            jax.ShapeDtypeStruct((2, 2 * D), jnp.float32),
            jax.ShapeDtypeStruct((1, 1), jnp.float32),
        ],
    )(query_address, lq, a2)
    weighted = wsum[0, :D] + wsum[1, D:]
    return weighted / ssum[0, 0]


# speedup vs baseline: 1.5902x; 1.3243x over previous
import jax
import jax.numpy as jnp
from jax.experimental import pallas as pl

N_ADDR = 1000000
D = 64
TEMPERATURE = 0.1
EPS = 1e-8

BLOCK = 8000


def _body(q_ref, a_ref, wsum_ref, ssum_ref):
    i = pl.program_id(0)

    @pl.when(i == 0)
    def _init():
        wsum_ref[...] = jnp.zeros_like(wsum_ref)
        ssum_ref[...] = jnp.zeros_like(ssum_ref)

    a = a_ref[...]
    q = q_ref[...]

    dots = jax.lax.dot_general(
        q, a, (((1,), (1,)), ((), ())),
        preferred_element_type=jnp.float32)
    ones = jnp.ones((1, D), jnp.float32)
    sumsq = jax.lax.dot_general(
        ones, a * a, (((1,), (1,)), ((), ())),
        preferred_element_type=jnp.float32)

    qn = jnp.sqrt(jnp.sum(q * q))
    an = jnp.sqrt(sumsq)
    sim = dots / jnp.maximum(an * qn, EPS)
    w = jnp.exp((sim - 1.0) / TEMPERATURE)

    part = jax.lax.dot_general(
        w, a, (((1,), (0,)), ((), ())),
        preferred_element_type=jnp.float32)
    wsum_ref[...] += part
    ssum_ref[...] += jnp.sum(w)


@jax.jit
def kernel(query_address, addresses):
    grid = N_ADDR // BLOCK
    wsum, ssum = pl.pallas_call(
        _body,
        grid=(grid,),
        in_specs=[
            pl.BlockSpec((1, D), lambda i: (0, 0)),
            pl.BlockSpec((BLOCK, D), lambda i: (i, 0)),
        ],
        out_specs=[
            pl.BlockSpec((1, D), lambda i: (0, 0)),
            pl.BlockSpec((1, 1), lambda i: (0, 0)),
        ],
        out_shape=[
            jax.ShapeDtypeStruct((1, D), jnp.float32),
            jax.ShapeDtypeStruct((1, 1), jnp.float32),
        ],
    )(query_address, addresses)
    return (wsum[0] / ssum[0, 0]).reshape(-1)


# speedup vs baseline: 1.7242x; 1.0843x over previous
import jax
import jax.numpy as jnp
from jax.experimental import pallas as pl
from jax.experimental.pallas import tpu as pltpu

N_ADDR = 1000000
D = 64
TEMPERATURE = 0.1
EPS = 1e-8

CHUNK = 8000
NBUF = 8
NCHUNK = N_ADDR // CHUNK


def _copy(a_hbm, buf, sem, c):
    return pltpu.make_async_copy(
        a_hbm.at[pl.ds(c * CHUNK, CHUNK), :], buf, sem)


def _body(q_ref, a_hbm, wsum_ref, ssum_ref, bufs, sems):
    i = pl.program_id(0)

    @pl.when(i == 0)
    def _prime():
        wsum_ref[...] = jnp.zeros_like(wsum_ref)
        ssum_ref[...] = jnp.zeros_like(ssum_ref)
        for k in range(NBUF - 1):
            _copy(a_hbm, bufs.at[k], sems.at[k], k).start()

    nxt = i + NBUF - 1

    @pl.when(nxt < NCHUNK)
    def _ahead():
        _copy(a_hbm, bufs.at[nxt % NBUF], sems.at[nxt % NBUF], nxt).start()

    _copy(a_hbm, bufs.at[i % NBUF], sems.at[i % NBUF], i).wait()
    a = bufs[i % NBUF]
    q = q_ref[...]

    dots = jax.lax.dot_general(
        q, a, (((1,), (1,)), ((), ())),
        preferred_element_type=jnp.float32)
    ones = jnp.ones((1, D), jnp.float32)
    sumsq = jax.lax.dot_general(
        ones, a * a, (((1,), (1,)), ((), ())),
        preferred_element_type=jnp.float32)

    qn = jnp.sqrt(jnp.sum(q * q))
    an = jnp.sqrt(sumsq)
    sim = dots / jnp.maximum(an * qn, EPS)
    w = jnp.exp((sim - 1.0) / TEMPERATURE)

    part = jax.lax.dot_general(
        w, a, (((1,), (0,)), ((), ())),
        preferred_element_type=jnp.float32)
    wsum_ref[...] += part
    ssum_ref[...] += jnp.sum(w)


@jax.jit
def kernel(query_address, addresses):
    wsum, ssum = pl.pallas_call(
        _body,
        grid=(NCHUNK,),
        in_specs=[
            pl.BlockSpec((1, D), lambda i: (0, 0)),
            pl.BlockSpec(memory_space=pl.ANY),
        ],
        out_specs=[
            pl.BlockSpec((1, D), lambda i: (0, 0)),
            pl.BlockSpec((1, 1), lambda i: (0, 0)),
        ],
        out_shape=[
            jax.ShapeDtypeStruct((1, D), jnp.float32),
            jax.ShapeDtypeStruct((1, 1), jnp.float32),
        ],
        scratch_shapes=[
            pltpu.VMEM((NBUF, CHUNK, D), jnp.float32),
            pltpu.SemaphoreType.DMA((NBUF,)),
        ],
    )(query_address, addresses)
    return (wsum[0] / ssum[0, 0]).reshape(-1)
